# baseline (speedup 1.0000x reference)
import functools

import jax
import jax.numpy as jnp
from jax import lax
from jax.experimental import pallas as pl
from jax.experimental.pallas import tpu as pltpu
from jax.experimental.pallas import tpu_sc as plsc

N = 10000
E = 160000
D_IN = 256
D_H = 64
NC = 2
NS = 16
NW = NC * NS
EPW = E // NW
CHUNK = 125
NCHUNK = EPW // CHUNK
ROWS_PS = N // NS
BLK = 2000
GRID = N // BLK
NBUF = 8

_SC_MESH = plsc.VectorSubcoreMesh(core_axis_name="c", subcore_axis_name="s")
_SC_PARAMS = pltpu.CompilerParams(
    needs_layout_passes=False, use_tc_tiling_on_sc=False)


@functools.partial(
    pl.kernel,
    out_type=jax.ShapeDtypeStruct((GRID, NW, BLK), jnp.float32),
    mesh=_SC_MESH,
    compiler_params=_SC_PARAMS,
    scratch_types=[
        pltpu.VMEM((EPW + 16,), jnp.int32),
        pltpu.VMEM((N,), jnp.float32),
    ],
)
def _deg_kernel(ei_hbm, out_hbm, idx_v, hist_v):
    c = lax.axis_index("c")
    s = lax.axis_index("s")
    w = c * NS + s
    zeros16 = jnp.zeros((16,), jnp.float32)

    def zero_body(i, carry):
        hist_v[pl.ds(i * 16, 16)] = zeros16
        return carry

    lax.fori_loop(0, N // 16, zero_body, 0)
    idx_v[pl.ds(EPW, 16)] = jnp.zeros((16,), jnp.int32)
    pltpu.sync_copy(ei_hbm.at[1, pl.ds(w * EPW, EPW)], idx_v.at[pl.ds(0, EPW)])
    ones16 = jnp.ones((16,), jnp.float32)

    def scat_body(k, carry):
        idx16 = idx_v[pl.ds(k * 16, 16)]
        plsc.addupdate_scatter(hist_v, [idx16], ones16)
        return carry

    lax.fori_loop(0, EPW // 16, scat_body, 0)
    rem = EPW - (EPW // 16) * 16
    if rem:
        lane = lax.iota(jnp.int32, 16)
        idx16 = idx_v[pl.ds((EPW // 16) * 16, 16)]
        plsc.addupdate_scatter(hist_v, [idx16], ones16, mask=lane < rem)
    for t in range(GRID):
        pltpu.sync_copy(hist_v.at[pl.ds(t * BLK, BLK)], out_hbm.at[t, w])


@functools.partial(
    pl.kernel,
    out_type=jax.ShapeDtypeStruct((NC, NS, ROWS_PS, D_H), jnp.float32),
    mesh=_SC_MESH,
    compiler_params=_SC_PARAMS,
    scratch_types=[
        pltpu.VMEM((NCHUNK, CHUNK), jnp.int32),
        pltpu.VMEM((NCHUNK, CHUNK), jnp.int32),
        [pltpu.VMEM((CHUNK, D_H), jnp.float32) for _ in range(NBUF)],
        pltpu.VMEM_SHARED((N, D_H), jnp.float32),
        [pltpu.SemaphoreType.DMA for _ in range(NBUF)],
        [pltpu.SemaphoreType.DMA for _ in range(NBUF)],
    ],
)
def _agg_kernel(g_hbm, er_hbm, out_hbm, src_v, dst_v, rows, acc_sh, gsem,
                ssem):
    c = lax.axis_index("c")
    s = lax.axis_index("s")
    w = c * NS + s
    pltpu.sync_copy(er_hbm.at[0, w], src_v)
    pltpu.sync_copy(er_hbm.at[1, w], dst_v)
    zeros16 = jnp.zeros((16,), jnp.float32)

    def zb(i, carry):
        r = i // (D_H // 16)
        k = i % (D_H // 16)
        rows[0][r, pl.ds(k * 16, 16)] = zeros16
        return carry

    lax.fori_loop(0, CHUNK * (D_H // 16), zb, 0)
    for t in range(ROWS_PS // CHUNK):
        pltpu.sync_copy(rows[0], acc_sh.at[pl.ds(s * ROWS_PS + t * CHUNK, CHUNK)])
    plsc.subcore_barrier()

    for b in range(NBUF):
        pltpu.async_copy(g_hbm.at[src_v.at[b]], rows[b], gsem[b])

    def group_body(p, carry):
        j0 = NBUF * p
        for b in range(NBUF):
            pltpu.make_async_copy(g_hbm.at[src_v.at[j0 + b]], rows[b],
                                  gsem[b]).wait()
            pltpu.async_copy(rows[b], acc_sh.at[dst_v.at[j0 + b]], ssem[b],
                             add=True)
        for b in range(NBUF):
            pltpu.make_async_copy(rows[b], acc_sh.at[dst_v.at[j0 + b]],
                                  ssem[b]).wait()
            pltpu.async_copy(g_hbm.at[src_v.at[j0 + NBUF + b]], rows[b],
                             gsem[b])
        return carry

    lax.fori_loop(0, NCHUNK // NBUF - 1, group_body, 0)
    j0 = NCHUNK - NBUF
    for b in range(NBUF):
        pltpu.make_async_copy(g_hbm.at[src_v.at[j0 + b]], rows[b],
                              gsem[b]).wait()
        pltpu.async_copy(rows[b], acc_sh.at[dst_v.at[j0 + b]], ssem[b],
                         add=True)
    for b in range(NBUF):
        pltpu.make_async_copy(rows[b], acc_sh.at[dst_v.at[j0 + b]],
                              ssem[b]).wait()
    plsc.subcore_barrier()
    pltpu.sync_copy(acc_sh.at[pl.ds(s * ROWS_PS, ROWS_PS)], out_hbm.at[c, s])


def _k1a_body(x_ref, w_ref, o_ref):
    o_ref[...] = jnp.dot(x_ref[...], w_ref[...],
                         preferred_element_type=jnp.float32)


def _k1b_body(h_ref, degp_ref, o_ref, d_ref):
    deg = jnp.sum(degp_ref[0], axis=0) + 1.0
    dinv = lax.rsqrt(deg)
    o_ref[...] = h_ref[...] * dinv[:, None]
    d_ref[0, 0] = dinv


def _k3_body(acc_ref, g_ref, d_ref, w_ref, b_ref, o_ref):
    dinv = d_ref[0, 0]
    tot = (acc_ref[0] + acc_ref[1] + g_ref[...]) * dinv[:, None] + b_ref[...]
    z = jnp.maximum(tot, 0.0)
    h = jnp.dot(z, w_ref[...], preferred_element_type=jnp.float32)
    o_ref[...] = h * dinv[:, None]


def _k5_body(acc_ref, g_ref, d_ref, b_ref, o_ref):
    i = pl.program_id(0)
    dinv = d_ref[0, 0]
    tot = (acc_ref[0] + acc_ref[1] + g_ref[...]) * dinv[:, None] + b_ref[...]
    z = jnp.maximum(tot, 0.0)
    p = jnp.sum(z, axis=0, keepdims=True)
    prev = jnp.where(i == 0, jnp.zeros_like(p), o_ref[...])
    accum = prev + p
    o_ref[...] = jnp.where(i == GRID - 1, accum * (1.0 / N), accum)


def _matmul1(x, W1):
    return pl.pallas_call(
        _k1a_body,
        grid=(GRID,),
        in_specs=[
            pl.BlockSpec((BLK, D_IN), lambda i: (i, 0)),
            pl.BlockSpec((D_IN, D_H), lambda i: (0, 0)),
        ],
        out_specs=pl.BlockSpec((BLK, D_H), lambda i: (i, 0)),
        out_shape=jax.ShapeDtypeStruct((N, D_H), jnp.float32),
    )(x, W1)


def _scale1(h1, degp):
    return pl.pallas_call(
        _k1b_body,
        grid=(GRID,),
        in_specs=[
            pl.BlockSpec((BLK, D_H), lambda i: (i, 0)),
            pl.BlockSpec((1, NW, BLK), lambda i: (i, 0, 0)),
        ],
        out_specs=[
            pl.BlockSpec((BLK, D_H), lambda i: (i, 0)),
            pl.BlockSpec((1, 1, BLK), lambda i: (i, 0, 0)),
        ],
        out_shape=[
            jax.ShapeDtypeStruct((N, D_H), jnp.float32),
            jax.ShapeDtypeStruct((GRID, 1, BLK), jnp.float32),
        ],
    )(h1, degp)


def _layer2(acc, g1, dinv3, W2, b1):
    return pl.pallas_call(
        _k3_body,
        grid=(GRID,),
        in_specs=[
            pl.BlockSpec((2, BLK, D_H), lambda i: (0, i, 0)),
            pl.BlockSpec((BLK, D_H), lambda i: (i, 0)),
            pl.BlockSpec((1, 1, BLK), lambda i: (i, 0, 0)),
            pl.BlockSpec((D_H, D_H), lambda i: (0, 0)),
            pl.BlockSpec((1, D_H), lambda i: (0, 0)),
        ],
        out_specs=pl.BlockSpec((BLK, D_H), lambda i: (i, 0)),
        out_shape=jax.ShapeDtypeStruct((N, D_H), jnp.float32),
    )(acc, g1, dinv3, W2, b1)


def _finalize(acc, g2, dinv3, b2):
    return pl.pallas_call(
        _k5_body,
        grid=(GRID,),
        in_specs=[
            pl.BlockSpec((2, BLK, D_H), lambda i: (0, i, 0)),
            pl.BlockSpec((BLK, D_H), lambda i: (i, 0)),
            pl.BlockSpec((1, 1, BLK), lambda i: (i, 0, 0)),
            pl.BlockSpec((1, D_H), lambda i: (0, 0)),
        ],
        out_specs=pl.BlockSpec((1, D_H), lambda i: (0, 0)),
        out_shape=jax.ShapeDtypeStruct((1, D_H), jnp.float32),
    )(acc, g2, dinv3, b2)


def kernel(x, edge_index, W1, b1, W2, b2):
    ei = edge_index.astype(jnp.int32)
    er = ei.reshape(2, NW, NCHUNK, CHUNK)
    b1r = b1.reshape(1, D_H).astype(jnp.float32)
    b2r = b2.reshape(1, D_H).astype(jnp.float32)

    degp = _deg_kernel(ei)
    h1 = _matmul1(x, W1)
    g1, dinv3 = _scale1(h1, degp)
    acc1 = _agg_kernel(g1, er).reshape(NC, N, D_H)
    g2 = _layer2(acc1, g1, dinv3, W2, b1r)
    acc2 = _agg_kernel(g2, er).reshape(NC, N, D_H)
    return _finalize(acc2, g2, dinv3, b2r)


# speedup vs baseline: 1.0415x; 1.0415x over previous
import functools

import jax
import jax.numpy as jnp
from jax import lax
from jax.experimental import pallas as pl
from jax.experimental.pallas import tpu as pltpu
from jax.experimental.pallas import tpu_sc as plsc

N = 10000
E = 160000
D_IN = 256
D_H = 64
NC = 2
NS = 16
NW = NC * NS
EPW = E // NW
CHUNK = 125
NCHUNK = EPW // CHUNK
ROWS_PS = N // NS
BLK = 2000
GRID = N // BLK
NBUF = 10

_SC_MESH = plsc.VectorSubcoreMesh(core_axis_name="c", subcore_axis_name="s")
_SC_PARAMS = pltpu.CompilerParams(
    needs_layout_passes=False, use_tc_tiling_on_sc=False)


@functools.partial(
    pl.kernel,
    out_type=jax.ShapeDtypeStruct((GRID, NW, BLK), jnp.float32),
    mesh=_SC_MESH,
    compiler_params=_SC_PARAMS,
    scratch_types=[
        pltpu.VMEM((EPW + 16,), jnp.int32),
        pltpu.VMEM((N,), jnp.float32),
    ],
)
def _deg_kernel(ei_hbm, out_hbm, idx_v, hist_v):
    c = lax.axis_index("c")
    s = lax.axis_index("s")
    w = c * NS + s
    zeros16 = jnp.zeros((16,), jnp.float32)

    def zero_body(i, carry):
        hist_v[pl.ds(i * 16, 16)] = zeros16
        return carry

    lax.fori_loop(0, N // 16, zero_body, 0)
    idx_v[pl.ds(EPW, 16)] = jnp.zeros((16,), jnp.int32)
    pltpu.sync_copy(ei_hbm.at[1, pl.ds(w * EPW, EPW)], idx_v.at[pl.ds(0, EPW)])
    ones16 = jnp.ones((16,), jnp.float32)

    def scat_body(k, carry):
        idx16 = idx_v[pl.ds(k * 16, 16)]
        plsc.addupdate_scatter(hist_v, [idx16], ones16)
        return carry

    lax.fori_loop(0, EPW // 16, scat_body, 0)
    rem = EPW - (EPW // 16) * 16
    if rem:
        lane = lax.iota(jnp.int32, 16)
        idx16 = idx_v[pl.ds((EPW // 16) * 16, 16)]
        plsc.addupdate_scatter(hist_v, [idx16], ones16, mask=lane < rem)
    for t in range(GRID):
        pltpu.sync_copy(hist_v.at[pl.ds(t * BLK, BLK)], out_hbm.at[t, w])


@functools.partial(
    pl.kernel,
    out_type=jax.ShapeDtypeStruct((NC, NS, ROWS_PS, D_H), jnp.float32),
    mesh=_SC_MESH,
    compiler_params=_SC_PARAMS,
    scratch_types=[
        pltpu.VMEM((NCHUNK, CHUNK), jnp.int32),
        pltpu.VMEM((NCHUNK, CHUNK), jnp.int32),
        [pltpu.VMEM((CHUNK, D_H), jnp.float32) for _ in range(NBUF)],
        pltpu.VMEM_SHARED((N, D_H), jnp.float32),
        [pltpu.SemaphoreType.DMA for _ in range(NBUF)],
        [pltpu.SemaphoreType.DMA for _ in range(NBUF)],
    ],
)
def _agg_kernel(g_hbm, er_hbm, out_hbm, src_v, dst_v, rows, acc_sh, gsem,
                ssem):
    c = lax.axis_index("c")
    s = lax.axis_index("s")
    w = c * NS + s
    pltpu.sync_copy(er_hbm.at[0, w], src_v)
    pltpu.sync_copy(er_hbm.at[1, w], dst_v)
    zeros16 = jnp.zeros((16,), jnp.float32)

    def zb(i, carry):
        r = i // (D_H // 16)
        k = i % (D_H // 16)
        rows[0][r, pl.ds(k * 16, 16)] = zeros16
        return carry

    lax.fori_loop(0, CHUNK * (D_H // 16), zb, 0)
    for t in range(ROWS_PS // CHUNK):
        pltpu.sync_copy(rows[0], acc_sh.at[pl.ds(s * ROWS_PS + t * CHUNK, CHUNK)])
    plsc.subcore_barrier()

    for b in range(NBUF):
        pltpu.async_copy(g_hbm.at[src_v.at[b]], rows[b], gsem[b])

    def group_body(p, carry):
        j0 = NBUF * p
        for b in range(NBUF):
            pltpu.make_async_copy(g_hbm.at[src_v.at[j0 + b]], rows[b],
                                  gsem[b]).wait()
            pltpu.async_copy(rows[b], acc_sh.at[dst_v.at[j0 + b]], ssem[b],
                             add=True)
        for b in range(NBUF):
            pltpu.make_async_copy(rows[b], acc_sh.at[dst_v.at[j0 + b]],
                                  ssem[b]).wait()
            pltpu.async_copy(g_hbm.at[src_v.at[j0 + NBUF + b]], rows[b],
                             gsem[b])
        return carry

    lax.fori_loop(0, NCHUNK // NBUF - 1, group_body, 0)
    j0 = NCHUNK - NBUF
    for b in range(NBUF):
        pltpu.make_async_copy(g_hbm.at[src_v.at[j0 + b]], rows[b],
                              gsem[b]).wait()
        pltpu.async_copy(rows[b], acc_sh.at[dst_v.at[j0 + b]], ssem[b],
                         add=True)
    for b in range(NBUF):
        pltpu.make_async_copy(rows[b], acc_sh.at[dst_v.at[j0 + b]],
                              ssem[b]).wait()
    plsc.subcore_barrier()
    pltpu.sync_copy(acc_sh.at[pl.ds(s * ROWS_PS, ROWS_PS)], out_hbm.at[c, s])


def _k1_body(x_ref, w_ref, degp_ref, o_ref, d_ref):
    deg = jnp.sum(degp_ref[0], axis=0) + 1.0
    dinv = lax.rsqrt(deg)
    h = jnp.dot(x_ref[...], w_ref[...], preferred_element_type=jnp.float32)
    o_ref[...] = h * dinv[:, None]
    d_ref[0, 0] = dinv


def _k3_body(acc_ref, g_ref, d_ref, w_ref, b_ref, o_ref):
    dinv = d_ref[0, 0]
    tot = (acc_ref[0] + acc_ref[1] + g_ref[...]) * dinv[:, None] + b_ref[...]
    z = jnp.maximum(tot, 0.0)
    h = jnp.dot(z, w_ref[...], preferred_element_type=jnp.float32)
    o_ref[...] = h * dinv[:, None]


def _k5_body(acc_ref, g_ref, d_ref, b_ref, o_ref):
    i = pl.program_id(0)
    dinv = d_ref[0, 0]
    tot = (acc_ref[0] + acc_ref[1] + g_ref[...]) * dinv[:, None] + b_ref[...]
    z = jnp.maximum(tot, 0.0)
    p = jnp.sum(z, axis=0, keepdims=True)
    prev = jnp.where(i == 0, jnp.zeros_like(p), o_ref[...])
    accum = prev + p
    o_ref[...] = jnp.where(i == GRID - 1, accum * (1.0 / N), accum)


def _scale_matmul(x, W1, degp):
    return pl.pallas_call(
        _k1_body,
        grid=(GRID,),
        in_specs=[
            pl.BlockSpec((BLK, D_IN), lambda i: (i, 0)),
            pl.BlockSpec((D_IN, D_H), lambda i: (0, 0)),
            pl.BlockSpec((1, NW, BLK), lambda i: (i, 0, 0)),
        ],
        out_specs=[
            pl.BlockSpec((BLK, D_H), lambda i: (i, 0)),
            pl.BlockSpec((1, 1, BLK), lambda i: (i, 0, 0)),
        ],
        out_shape=[
            jax.ShapeDtypeStruct((N, D_H), jnp.float32),
            jax.ShapeDtypeStruct((GRID, 1, BLK), jnp.float32),
        ],
    )(x, W1, degp)


def _layer2(acc, g1, dinv3, W2, b1):
    return pl.pallas_call(
        _k3_body,
        grid=(GRID,),
        in_specs=[
            pl.BlockSpec((2, BLK, D_H), lambda i: (0, i, 0)),
            pl.BlockSpec((BLK, D_H), lambda i: (i, 0)),
            pl.BlockSpec((1, 1, BLK), lambda i: (i, 0, 0)),
            pl.BlockSpec((D_H, D_H), lambda i: (0, 0)),
            pl.BlockSpec((1, D_H), lambda i: (0, 0)),
        ],
        out_specs=pl.BlockSpec((BLK, D_H), lambda i: (i, 0)),
        out_shape=jax.ShapeDtypeStruct((N, D_H), jnp.float32),
    )(acc, g1, dinv3, W2, b1)


def _finalize(acc, g2, dinv3, b2):
    return pl.pallas_call(
        _k5_body,
        grid=(GRID,),
        in_specs=[
            pl.BlockSpec((2, BLK, D_H), lambda i: (0, i, 0)),
            pl.BlockSpec((BLK, D_H), lambda i: (i, 0)),
            pl.BlockSpec((1, 1, BLK), lambda i: (i, 0, 0)),
            pl.BlockSpec((1, D_H), lambda i: (0, 0)),
        ],
        out_specs=pl.BlockSpec((1, D_H), lambda i: (0, 0)),
        out_shape=jax.ShapeDtypeStruct((1, D_H), jnp.float32),
    )(acc, g2, dinv3, b2)


def kernel(x, edge_index, W1, b1, W2, b2):
    ei = edge_index.astype(jnp.int32)
    er = ei.reshape(2, NW, NCHUNK, CHUNK)
    b1r = b1.reshape(1, D_H).astype(jnp.float32)
    b2r = b2.reshape(1, D_H).astype(jnp.float32)

    degp = _deg_kernel(ei)
    g1, dinv3 = _scale_matmul(x, W1, degp)
    acc1 = _agg_kernel(g1, er).reshape(NC, N, D_H)
    g2 = _layer2(acc1, g1, dinv3, W2, b1r)
    acc2 = _agg_kernel(g2, er).reshape(NC, N, D_H)
    return _finalize(acc2, g2, dinv3, b2r)


# speedup vs baseline: 1.0714x; 1.0286x over previous
import functools

import jax
import jax.numpy as jnp
from jax import lax
from jax.experimental import pallas as pl
from jax.experimental.pallas import tpu as pltpu
from jax.experimental.pallas import tpu_sc as plsc

N = 10000
E = 160000
D_IN = 256
D_H = 64
NC = 2
NS = 16
NW = NC * NS
EPW = E // NW
CHUNK = 125
NCHUNK = EPW // CHUNK
ROWS_PS = N // NS
BLK = 2000
GRID = N // BLK
NBUF = 10

_SC_MESH = plsc.VectorSubcoreMesh(core_axis_name="c", subcore_axis_name="s")
_SC_PARAMS = pltpu.CompilerParams(
    needs_layout_passes=False, use_tc_tiling_on_sc=False)


@functools.partial(
    pl.kernel,
    out_type=jax.ShapeDtypeStruct((GRID, NW, BLK), jnp.float32),
    mesh=_SC_MESH,
    compiler_params=_SC_PARAMS,
    scratch_types=[
        pltpu.VMEM((EPW + 16,), jnp.int32),
        pltpu.VMEM((N,), jnp.float32),
    ],
)
def _deg_kernel(ei_hbm, out_hbm, idx_v, hist_v):
    c = lax.axis_index("c")
    s = lax.axis_index("s")
    w = c * NS + s
    zeros16 = jnp.zeros((16,), jnp.float32)

    def zero_body(i, carry):
        hist_v[pl.ds(i * 16, 16)] = zeros16
        return carry

    lax.fori_loop(0, N // 16, zero_body, 0)
    idx_v[pl.ds(EPW, 16)] = jnp.zeros((16,), jnp.int32)
    pltpu.sync_copy(ei_hbm.at[1, pl.ds(w * EPW, EPW)], idx_v.at[pl.ds(0, EPW)])
    ones16 = jnp.ones((16,), jnp.float32)

    def scat_body(k, carry):
        idx16 = idx_v[pl.ds(k * 16, 16)]
        plsc.addupdate_scatter(hist_v, [idx16], ones16)
        return carry

    lax.fori_loop(0, EPW // 16, scat_body, 0)
    rem = EPW - (EPW // 16) * 16
    if rem:
        lane = lax.iota(jnp.int32, 16)
        idx16 = idx_v[pl.ds((EPW // 16) * 16, 16)]
        plsc.addupdate_scatter(hist_v, [idx16], ones16, mask=lane < rem)
    for t in range(GRID):
        pltpu.sync_copy(hist_v.at[pl.ds(t * BLK, BLK)], out_hbm.at[t, w])


@functools.partial(
    pl.kernel,
    out_type=jax.ShapeDtypeStruct((NC, NS, ROWS_PS, D_H), jnp.float32),
    mesh=_SC_MESH,
    compiler_params=_SC_PARAMS,
    scratch_types=[
        pltpu.VMEM((NCHUNK, CHUNK), jnp.int32),
        pltpu.VMEM((NCHUNK, CHUNK), jnp.int32),
        [pltpu.VMEM((CHUNK, D_H), jnp.float32) for _ in range(NBUF)],
        pltpu.VMEM_SHARED((N, D_H), jnp.float32),
        [pltpu.SemaphoreType.DMA for _ in range(NBUF)],
        [pltpu.SemaphoreType.DMA for _ in range(NBUF)],
    ],
)
def _agg_kernel(g_hbm, er_hbm, out_hbm, src_v, dst_v, rows, acc_sh, gsem,
                ssem):
    c = lax.axis_index("c")
    s = lax.axis_index("s")
    w = c * NS + s
    pltpu.async_copy(er_hbm.at[0, w], src_v, gsem[0])
    pltpu.async_copy(er_hbm.at[1, w], dst_v, gsem[1])
    zeros16 = jnp.zeros((16,), jnp.float32)

    def zb(i, carry):
        r = i // (D_H // 16)
        k = i % (D_H // 16)
        rows[0][r, pl.ds(k * 16, 16)] = zeros16
        return carry

    lax.fori_loop(0, CHUNK * (D_H // 16), zb, 0)
    nz = ROWS_PS // CHUNK
    for t in range(nz):
        pltpu.async_copy(rows[0],
                         acc_sh.at[pl.ds(s * ROWS_PS + t * CHUNK, CHUNK)],
                         ssem[t])
    for t in range(nz):
        pltpu.make_async_copy(rows[0],
                              acc_sh.at[pl.ds(s * ROWS_PS + t * CHUNK, CHUNK)],
                              ssem[t]).wait()
    pltpu.make_async_copy(er_hbm.at[0, w], src_v, gsem[0]).wait()
    pltpu.make_async_copy(er_hbm.at[1, w], dst_v, gsem[1]).wait()
    plsc.subcore_barrier()

    for b in range(NBUF):
        pltpu.async_copy(g_hbm.at[src_v.at[b]], rows[b], gsem[b])

    def group_body(p, carry):
        j0 = NBUF * p
        for b in range(NBUF):
            pltpu.make_async_copy(g_hbm.at[src_v.at[j0 + b]], rows[b],
                                  gsem[b]).wait()
            pltpu.async_copy(rows[b], acc_sh.at[dst_v.at[j0 + b]], ssem[b],
                             add=True)
        for b in range(NBUF):
            pltpu.make_async_copy(rows[b], acc_sh.at[dst_v.at[j0 + b]],
                                  ssem[b]).wait()
            pltpu.async_copy(g_hbm.at[src_v.at[j0 + NBUF + b]], rows[b],
                             gsem[b])
        return carry

    lax.fori_loop(0, NCHUNK // NBUF - 1, group_body, 0)
    j0 = NCHUNK - NBUF
    for b in range(NBUF):
        pltpu.make_async_copy(g_hbm.at[src_v.at[j0 + b]], rows[b],
                              gsem[b]).wait()
        pltpu.async_copy(rows[b], acc_sh.at[dst_v.at[j0 + b]], ssem[b],
                         add=True)
    for b in range(NBUF):
        pltpu.make_async_copy(rows[b], acc_sh.at[dst_v.at[j0 + b]],
                              ssem[b]).wait()
    plsc.subcore_barrier()
    pltpu.sync_copy(acc_sh.at[pl.ds(s * ROWS_PS, ROWS_PS)], out_hbm.at[c, s])


def _k1_body(x_ref, w_ref, degp_ref, o_ref, d_ref):
    deg = jnp.sum(degp_ref[0], axis=0) + 1.0
    dinv = lax.rsqrt(deg)
    h = jnp.dot(x_ref[...], w_ref[...], preferred_element_type=jnp.float32)
    o_ref[...] = h * dinv[:, None]
    d_ref[0, 0] = dinv


def _k3_body(acc_ref, g_ref, d_ref, w_ref, b_ref, o_ref):
    dinv = d_ref[0, 0]
    tot = (acc_ref[0] + acc_ref[1] + g_ref[...]) * dinv[:, None] + b_ref[...]
    z = jnp.maximum(tot, 0.0)
    h = jnp.dot(z, w_ref[...], preferred_element_type=jnp.float32)
    o_ref[...] = h * dinv[:, None]


def _k5_body(acc_ref, g_ref, d_ref, b_ref, o_ref):
    i = pl.program_id(0)
    dinv = d_ref[0, 0]
    tot = (acc_ref[0] + acc_ref[1] + g_ref[...]) * dinv[:, None] + b_ref[...]
    z = jnp.maximum(tot, 0.0)
    p = jnp.sum(z, axis=0, keepdims=True)
    prev = jnp.where(i == 0, jnp.zeros_like(p), o_ref[...])
    accum = prev + p
    o_ref[...] = jnp.where(i == GRID - 1, accum * (1.0 / N), accum)


def _scale_matmul(x, W1, degp):
    return pl.pallas_call(
        _k1_body,
        grid=(GRID,),
        in_specs=[
            pl.BlockSpec((BLK, D_IN), lambda i: (i, 0)),
            pl.BlockSpec((D_IN, D_H), lambda i: (0, 0)),
            pl.BlockSpec((1, NW, BLK), lambda i: (i, 0, 0)),
        ],
        out_specs=[
            pl.BlockSpec((BLK, D_H), lambda i: (i, 0)),
            pl.BlockSpec((1, 1, BLK), lambda i: (i, 0, 0)),
        ],
        out_shape=[
            jax.ShapeDtypeStruct((N, D_H), jnp.float32),
            jax.ShapeDtypeStruct((GRID, 1, BLK), jnp.float32),
        ],
    )(x, W1, degp)


def _layer2(acc, g1, dinv3, W2, b1):
    return pl.pallas_call(
        _k3_body,
        grid=(GRID,),
        in_specs=[
            pl.BlockSpec((2, BLK, D_H), lambda i: (0, i, 0)),
            pl.BlockSpec((BLK, D_H), lambda i: (i, 0)),
            pl.BlockSpec((1, 1, BLK), lambda i: (i, 0, 0)),
            pl.BlockSpec((D_H, D_H), lambda i: (0, 0)),
            pl.BlockSpec((1, D_H), lambda i: (0, 0)),
        ],
        out_specs=pl.BlockSpec((BLK, D_H), lambda i: (i, 0)),
        out_shape=jax.ShapeDtypeStruct((N, D_H), jnp.float32),
    )(acc, g1, dinv3, W2, b1)


def _finalize(acc, g2, dinv3, b2):
    return pl.pallas_call(
        _k5_body,
        grid=(GRID,),
        in_specs=[
            pl.BlockSpec((2, BLK, D_H), lambda i: (0, i, 0)),
            pl.BlockSpec((BLK, D_H), lambda i: (i, 0)),
            pl.BlockSpec((1, 1, BLK), lambda i: (i, 0, 0)),
            pl.BlockSpec((1, D_H), lambda i: (0, 0)),
        ],
        out_specs=pl.BlockSpec((1, D_H), lambda i: (0, 0)),
        out_shape=jax.ShapeDtypeStruct((1, D_H), jnp.float32),
    )(acc, g2, dinv3, b2)


def kernel(x, edge_index, W1, b1, W2, b2):
    ei = edge_index.astype(jnp.int32)
    er = ei.reshape(2, NW, NCHUNK, CHUNK)
    b1r = b1.reshape(1, D_H).astype(jnp.float32)
    b2r = b2.reshape(1, D_H).astype(jnp.float32)

    degp = _deg_kernel(ei)
    g1, dinv3 = _scale_matmul(x, W1, degp)
    acc1 = _agg_kernel(g1, er).reshape(NC, N, D_H)
    g2 = _layer2(acc1, g1, dinv3, W2, b1r)
    acc2 = _agg_kernel(g2, er).reshape(NC, N, D_H)
    return _finalize(acc2, g2, dinv3, b2r)


# speedup vs baseline: 1.0773x; 1.0055x over previous
import functools

import jax
import jax.numpy as jnp
from jax import lax
from jax.experimental import pallas as pl
from jax.experimental.pallas import tpu as pltpu
from jax.experimental.pallas import tpu_sc as plsc

N = 10000
E = 160000
D_IN = 256
D_H = 64
NC = 2
NS = 16
NW = NC * NS
EPW = E // NW
CHUNK = 125
NCHUNK = EPW // CHUNK
ROWS_PS = N // NS
BLK = 2000
GRID = N // BLK
NBUF = 10

_SC_MESH = plsc.VectorSubcoreMesh(core_axis_name="c", subcore_axis_name="s")
_SC_PARAMS = pltpu.CompilerParams(
    needs_layout_passes=False, use_tc_tiling_on_sc=False)


@functools.partial(
    pl.kernel,
    out_type=jax.ShapeDtypeStruct((GRID, NW, BLK), jnp.float32),
    mesh=_SC_MESH,
    compiler_params=_SC_PARAMS,
    scratch_types=[
        pltpu.VMEM((EPW + 16,), jnp.int32),
        pltpu.VMEM((N,), jnp.float32),
        pltpu.SemaphoreType.DMA,
        [pltpu.SemaphoreType.DMA for _ in range(GRID)],
    ],
)
def _deg_kernel(ei_hbm, out_hbm, idx_v, hist_v, isem, osem):
    c = lax.axis_index("c")
    s = lax.axis_index("s")
    w = c * NS + s
    idx_v[pl.ds(EPW, 16)] = jnp.zeros((16,), jnp.int32)
    pltpu.async_copy(ei_hbm.at[1, pl.ds(w * EPW, EPW)], idx_v.at[pl.ds(0, EPW)],
                     isem)
    zeros16 = jnp.zeros((16,), jnp.float32)

    def zero_body(i, carry):
        hist_v[pl.ds(i * 16, 16)] = zeros16
        return carry

    lax.fori_loop(0, N // 16, zero_body, 0)
    pltpu.make_async_copy(ei_hbm.at[1, pl.ds(w * EPW, EPW)],
                          idx_v.at[pl.ds(0, EPW)], isem).wait()
    ones16 = jnp.ones((16,), jnp.float32)

    def scat_body(k, carry):
        idx16 = idx_v[pl.ds(k * 16, 16)]
        plsc.addupdate_scatter(hist_v, [idx16], ones16)
        return carry

    lax.fori_loop(0, EPW // 16, scat_body, 0)
    rem = EPW - (EPW // 16) * 16
    if rem:
        lane = lax.iota(jnp.int32, 16)
        idx16 = idx_v[pl.ds((EPW // 16) * 16, 16)]
        plsc.addupdate_scatter(hist_v, [idx16], ones16, mask=lane < rem)
    for t in range(GRID):
        pltpu.async_copy(hist_v.at[pl.ds(t * BLK, BLK)], out_hbm.at[t, w],
                         osem[t])
    for t in range(GRID):
        pltpu.make_async_copy(hist_v.at[pl.ds(t * BLK, BLK)], out_hbm.at[t, w],
                              osem[t]).wait()


@functools.partial(
    pl.kernel,
    out_type=jax.ShapeDtypeStruct((NC, NS, ROWS_PS, D_H), jnp.float32),
    mesh=_SC_MESH,
    compiler_params=_SC_PARAMS,
    scratch_types=[
        pltpu.VMEM((NCHUNK, CHUNK), jnp.int32),
        pltpu.VMEM((NCHUNK, CHUNK), jnp.int32),
        [pltpu.VMEM((CHUNK, D_H), jnp.float32) for _ in range(NBUF)],
        pltpu.VMEM_SHARED((N, D_H), jnp.float32),
        [pltpu.SemaphoreType.DMA for _ in range(NBUF)],
        [pltpu.SemaphoreType.DMA for _ in range(NBUF)],
    ],
)
def _agg_kernel(g_hbm, er_hbm, out_hbm, src_v, dst_v, rows, acc_sh, gsem,
                ssem):
    c = lax.axis_index("c")
    s = lax.axis_index("s")
    w = c * NS + s
    pltpu.async_copy(er_hbm.at[0, w], src_v, gsem[0])
    pltpu.async_copy(er_hbm.at[1, w], dst_v, gsem[1])
    zeros16 = jnp.zeros((16,), jnp.float32)

    def zb(i, carry):
        r = i // (D_H // 16)
        k = i % (D_H // 16)
        rows[0][r, pl.ds(k * 16, 16)] = zeros16
        return carry

    lax.fori_loop(0, CHUNK * (D_H // 16), zb, 0)
    nz = ROWS_PS // CHUNK
    for t in range(nz):
        pltpu.async_copy(rows[0],
                         acc_sh.at[pl.ds(s * ROWS_PS + t * CHUNK, CHUNK)],
                         ssem[t])
    for t in range(nz):
        pltpu.make_async_copy(rows[0],
                              acc_sh.at[pl.ds(s * ROWS_PS + t * CHUNK, CHUNK)],
                              ssem[t]).wait()
    pltpu.make_async_copy(er_hbm.at[0, w], src_v, gsem[0]).wait()
    pltpu.make_async_copy(er_hbm.at[1, w], dst_v, gsem[1]).wait()
    plsc.subcore_barrier()

    for b in range(NBUF):
        pltpu.async_copy(g_hbm.at[src_v.at[b]], rows[b], gsem[b])

    def group_body(p, carry):
        j0 = NBUF * p
        for b in range(NBUF):
            pltpu.make_async_copy(g_hbm.at[src_v.at[j0 + b]], rows[b],
                                  gsem[b]).wait()
            pltpu.async_copy(rows[b], acc_sh.at[dst_v.at[j0 + b]], ssem[b],
                             add=True)
        for b in range(NBUF):
            pltpu.make_async_copy(rows[b], acc_sh.at[dst_v.at[j0 + b]],
                                  ssem[b]).wait()
            pltpu.async_copy(g_hbm.at[src_v.at[j0 + NBUF + b]], rows[b],
                             gsem[b])
        return carry

    lax.fori_loop(0, NCHUNK // NBUF - 1, group_body, 0)
    j0 = NCHUNK - NBUF
    for b in range(NBUF):
        pltpu.make_async_copy(g_hbm.at[src_v.at[j0 + b]], rows[b],
                              gsem[b]).wait()
        pltpu.async_copy(rows[b], acc_sh.at[dst_v.at[j0 + b]], ssem[b],
                         add=True)
    for b in range(NBUF):
        pltpu.make_async_copy(rows[b], acc_sh.at[dst_v.at[j0 + b]],
                              ssem[b]).wait()
    plsc.subcore_barrier()
    pltpu.sync_copy(acc_sh.at[pl.ds(s * ROWS_PS, ROWS_PS)], out_hbm.at[c, s])


def _k1_body(x_ref, w_ref, degp_ref, o_ref, d_ref):
    deg = jnp.sum(degp_ref[0], axis=0) + 1.0
    dinv = lax.rsqrt(deg)
    h = jnp.dot(x_ref[...], w_ref[...], preferred_element_type=jnp.float32)
    o_ref[...] = h * dinv[:, None]
    d_ref[0, 0] = dinv


def _k3_body(acc_ref, g_ref, d_ref, w_ref, b_ref, o_ref):
    dinv = d_ref[0, 0]
    tot = (acc_ref[0] + acc_ref[1] + g_ref[...]) * dinv[:, None] + b_ref[...]
    z = jnp.maximum(tot, 0.0)
    h = jnp.dot(z, w_ref[...], preferred_element_type=jnp.float32)
    o_ref[...] = h * dinv[:, None]


def _k5_body(acc_ref, g_ref, d_ref, b_ref, o_ref):
    i = pl.program_id(0)
    dinv = d_ref[0, 0]
    tot = (acc_ref[0] + acc_ref[1] + g_ref[...]) * dinv[:, None] + b_ref[...]
    z = jnp.maximum(tot, 0.0)
    p = jnp.sum(z, axis=0, keepdims=True)
    prev = jnp.where(i == 0, jnp.zeros_like(p), o_ref[...])
    accum = prev + p
    o_ref[...] = jnp.where(i == GRID - 1, accum * (1.0 / N), accum)


def _scale_matmul(x, W1, degp):
    return pl.pallas_call(
        _k1_body,
        grid=(GRID,),
        in_specs=[
            pl.BlockSpec((BLK, D_IN), lambda i: (i, 0)),
            pl.BlockSpec((D_IN, D_H), lambda i: (0, 0)),
            pl.BlockSpec((1, NW, BLK), lambda i: (i, 0, 0)),
        ],
        out_specs=[
            pl.BlockSpec((BLK, D_H), lambda i: (i, 0)),
            pl.BlockSpec((1, 1, BLK), lambda i: (i, 0, 0)),
        ],
        out_shape=[
            jax.ShapeDtypeStruct((N, D_H), jnp.float32),
            jax.ShapeDtypeStruct((GRID, 1, BLK), jnp.float32),
        ],
    )(x, W1, degp)


def _layer2(acc, g1, dinv3, W2, b1):
    return pl.pallas_call(
        _k3_body,
        grid=(GRID,),
        in_specs=[
            pl.BlockSpec((2, BLK, D_H), lambda i: (0, i, 0)),
            pl.BlockSpec((BLK, D_H), lambda i: (i, 0)),
            pl.BlockSpec((1, 1, BLK), lambda i: (i, 0, 0)),
            pl.BlockSpec((D_H, D_H), lambda i: (0, 0)),
            pl.BlockSpec((1, D_H), lambda i: (0, 0)),
        ],
        out_specs=pl.BlockSpec((BLK, D_H), lambda i: (i, 0)),
        out_shape=jax.ShapeDtypeStruct((N, D_H), jnp.float32),
    )(acc, g1, dinv3, W2, b1)


def _finalize(acc, g2, dinv3, b2):
    return pl.pallas_call(
        _k5_body,
        grid=(GRID,),
        in_specs=[
            pl.BlockSpec((2, BLK, D_H), lambda i: (0, i, 0)),
            pl.BlockSpec((BLK, D_H), lambda i: (i, 0)),
            pl.BlockSpec((1, 1, BLK), lambda i: (i, 0, 0)),
            pl.BlockSpec((1, D_H), lambda i: (0, 0)),
        ],
        out_specs=pl.BlockSpec((1, D_H), lambda i: (0, 0)),
        out_shape=jax.ShapeDtypeStruct((1, D_H), jnp.float32),
    )(acc, g2, dinv3, b2)


def kernel(x, edge_index, W1, b1, W2, b2):
    ei = edge_index.astype(jnp.int32)
    er = ei.reshape(2, NW, NCHUNK, CHUNK)
    b1r = b1.reshape(1, D_H).astype(jnp.float32)
    b2r = b2.reshape(1, D_H).astype(jnp.float32)

    degp = _deg_kernel(ei)
    g1, dinv3 = _scale_matmul(x, W1, degp)
    acc1 = _agg_kernel(g1, er).reshape(NC, N, D_H)
    g2 = _layer2(acc1, g1, dinv3, W2, b1r)
    acc2 = _agg_kernel(g2, er).reshape(NC, N, D_H)
    return _finalize(acc2, g2, dinv3, b2r)
